# baseline (speedup 1.0000x reference)
import jax
import jax.numpy as jnp
from jax.experimental import pallas as pl
from jax.experimental.pallas import tpu as pltpu

N = 8192
D = 128
BM = 128


def _bf16_dot(a, b):
    return jax.lax.dot_general(
        a.astype(jnp.bfloat16), b.astype(jnp.bfloat16),
        (((1,), (0,)), ((), ())), preferred_element_type=jnp.float32)


def _pipeline_block(sup, feat, w):
    agg = _bf16_dot(sup, feat)
    t = jnp.tanh(_bf16_dot(agg, w))
    nrm = jnp.sqrt(jnp.sum(t * t, axis=1, keepdims=True))
    return t / jnp.maximum(nrm, 1e-12)


def _gnn_block(sup_t_ref, sup_g_ref, feat_t_ref, feat_g_ref, w_ref,
               out_t_ref, out_g_ref):
    w = w_ref[...]
    out_t_ref[...] = _pipeline_block(sup_t_ref[...], feat_t_ref[...], w)
    out_g_ref[...] = _pipeline_block(sup_g_ref[...], feat_g_ref[...], w)


def kernel(feat_topo, sup_topo, feat_gnd, sup_gnd, train_flag, W):
    sup_spec = pl.BlockSpec((BM, N), lambda i: (i, 0))
    feat_spec = pl.BlockSpec((N, D), lambda i: (0, 0))
    out_spec = pl.BlockSpec((BM, D), lambda i: (i, 0))
    out_topo, out_gnd = pl.pallas_call(
        _gnn_block,
        grid=(N // BM,),
        in_specs=[sup_spec, sup_spec, feat_spec, feat_spec,
                  pl.BlockSpec((D, D), lambda i: (0, 0))],
        out_specs=[out_spec, out_spec],
        out_shape=[jax.ShapeDtypeStruct((N, D), jnp.float32),
                   jax.ShapeDtypeStruct((N, D), jnp.float32)],
        compiler_params=pltpu.CompilerParams(
            dimension_semantics=("parallel",),
        ),
    )(sup_topo, sup_gnd, feat_topo, feat_gnd, W)
    out_gnd = jnp.where(train_flag != 0, out_gnd, jnp.zeros_like(out_gnd))
    return (out_topo, out_gnd)


# speedup vs baseline: 1.0441x; 1.0441x over previous
import jax
import jax.numpy as jnp
from jax.experimental import pallas as pl
from jax.experimental.pallas import tpu as pltpu

N = 8192
D = 128
BM = 256


def _bf16_dot(a, b):
    return jax.lax.dot_general(
        a.astype(jnp.bfloat16), b.astype(jnp.bfloat16),
        (((1,), (0,)), ((), ())), preferred_element_type=jnp.float32)


def _pipeline_block(sup, feat, w):
    agg = _bf16_dot(sup, feat)
    t = jnp.tanh(_bf16_dot(agg, w))
    nrm = jnp.sqrt(jnp.sum(t * t, axis=1, keepdims=True))
    return t / jnp.maximum(nrm, 1e-12)


def _gnn_block(sup_t_ref, sup_g_ref, feat_t_ref, feat_g_ref, w_ref,
               out_t_ref, out_g_ref):
    w = w_ref[...]
    out_t_ref[...] = _pipeline_block(sup_t_ref[...], feat_t_ref[...], w)
    out_g_ref[...] = _pipeline_block(sup_g_ref[...], feat_g_ref[...], w)


def kernel(feat_topo, sup_topo, feat_gnd, sup_gnd, train_flag, W):
    sup_spec = pl.BlockSpec((BM, N), lambda i: (i, 0))
    feat_spec = pl.BlockSpec((N, D), lambda i: (0, 0))
    out_spec = pl.BlockSpec((BM, D), lambda i: (i, 0))
    out_topo, out_gnd = pl.pallas_call(
        _gnn_block,
        grid=(N // BM,),
        in_specs=[sup_spec, sup_spec, feat_spec, feat_spec,
                  pl.BlockSpec((D, D), lambda i: (0, 0))],
        out_specs=[out_spec, out_spec],
        out_shape=[jax.ShapeDtypeStruct((N, D), jnp.float32),
                   jax.ShapeDtypeStruct((N, D), jnp.float32)],
        compiler_params=pltpu.CompilerParams(
            dimension_semantics=("parallel",),
        ),
    )(sup_topo, sup_gnd, feat_topo, feat_gnd, W)
    out_gnd = jnp.where(train_flag != 0, out_gnd, jnp.zeros_like(out_gnd))
    return (out_topo, out_gnd)


# speedup vs baseline: 1.0707x; 1.0255x over previous
import jax
import jax.numpy as jnp
from jax.experimental import pallas as pl
from jax.experimental.pallas import tpu as pltpu

N = 8192
D = 128
BM = 256


def _bf16_dot(a, b):
    return jax.lax.dot_general(
        a.astype(jnp.bfloat16), b.astype(jnp.bfloat16),
        (((1,), (0,)), ((), ())), preferred_element_type=jnp.float32)


def _pipeline_block(sup, feat, w):
    agg = _bf16_dot(sup, feat)
    t = jnp.tanh(_bf16_dot(agg, w))
    nrm = jnp.sqrt(jnp.sum(t * t, axis=1, keepdims=True))
    return t / jnp.maximum(nrm, 1e-12)


def _gnn_block(flag_ref, sup_t_ref, sup_g_ref, feat_t_ref, feat_g_ref, w_ref,
               out_t_ref, out_g_ref):
    w = w_ref[...]
    out_t_ref[...] = _pipeline_block(sup_t_ref[...], feat_t_ref[...], w)
    out_g_ref[...] = _pipeline_block(sup_g_ref[...], feat_g_ref[...], w) \
        * flag_ref[0]


def kernel(feat_topo, sup_topo, feat_gnd, sup_gnd, train_flag, W):
    flag = jnp.where(jnp.asarray(train_flag) != 0, 1.0, 0.0).reshape(1)
    flag = flag.astype(jnp.float32)
    sup_spec = pl.BlockSpec((BM, N), lambda i: (i, 0))
    feat_spec = pl.BlockSpec((N, D), lambda i: (0, 0))
    out_spec = pl.BlockSpec((BM, D), lambda i: (i, 0))
    out_topo, out_gnd = pl.pallas_call(
        _gnn_block,
        grid=(N // BM,),
        in_specs=[pl.BlockSpec(memory_space=pltpu.MemorySpace.SMEM),
                  sup_spec, sup_spec, feat_spec, feat_spec,
                  pl.BlockSpec((D, D), lambda i: (0, 0))],
        out_specs=[out_spec, out_spec],
        out_shape=[jax.ShapeDtypeStruct((N, D), jnp.float32),
                   jax.ShapeDtypeStruct((N, D), jnp.float32)],
        compiler_params=pltpu.CompilerParams(
            dimension_semantics=("parallel",),
        ),
    )(flag, sup_topo, sup_gnd, feat_topo, feat_gnd, W)
    return (out_topo, out_gnd)
